# baseline (device time: 200598 ns/iter reference)
import functools

import jax
import jax.numpy as jnp
from jax import lax
from jax.experimental import pallas as pl
from jax.experimental.pallas import tpu as pltpu

N_DEV = 4


def kernel(x, k, Wp):
    B, S, C = x.shape
    KT = k.shape[0]
    P = Wp.shape[1]
    CHUNK = S // N_DEV
    SUB = 512
    HALO = 8

    def body(x_ref, k_ref, w_ref, out_ref,
             xw, cb, rb, ab, stage,
             send_sems, recv_sems, copy_sems, stage_sems, credit_sems):
        me = lax.axis_index("i")
        left = (me - 1) % N_DEV
        right = (me + 1) % N_DEV
        downstream = [right, left]
        upstream = [left, right]

        barrier_sem = pltpu.get_barrier_semaphore()
        for nbr in [left, right]:
            pl.semaphore_signal(
                barrier_sem, inc=1,
                device_id=(nbr,), device_id_type=pl.DeviceIdType.MESH,
            )
        pl.semaphore_wait(barrier_sem, 2)

        def load_start(r, cj):
            lo = cj * CHUNK

            @pl.when(cj > 0)
            def _():
                pltpu.make_async_copy(
                    x_ref.at[r, pl.ds(lo - HALO, CHUNK + HALO), :], xw.at[r],
                    copy_sems.at[r],
                ).start()

            @pl.when(cj == 0)
            def _():
                pltpu.make_async_copy(
                    x_ref.at[r, pl.ds(0, CHUNK), :],
                    xw.at[r, pl.ds(HALO, CHUNK), :],
                    copy_sems.at[r],
                ).start()
                xw[r, 0:HALO, :] = jnp.zeros((HALO, C), jnp.float32)

        def load_wait(r, cj):
            lo = cj * CHUNK

            @pl.when(cj > 0)
            def _():
                pltpu.make_async_copy(
                    x_ref.at[r, pl.ds(lo - HALO, CHUNK + HALO), :], xw.at[r],
                    copy_sems.at[r],
                ).wait()

            @pl.when(cj == 0)
            def _():
                pltpu.make_async_copy(
                    x_ref.at[r, pl.ds(0, CHUNK), :],
                    xw.at[r, pl.ds(HALO, CHUNK), :],
                    copy_sems.at[r],
                ).wait()

        def conv_silu_dot(r, u):
            us = u * SUB + HALO
            acc = xw[r, us:us + SUB, :] * k_ref[KT - 1, :][None, :]
            for t in range(KT - 1):
                sh = KT - 1 - t
                acc += xw[r, us - sh:us + SUB - sh, :] * k_ref[t, :][None, :]
            a = acc * (1.0 / (1.0 + jnp.exp(-acc)))
            return jnp.dot(
                a.astype(jnp.bfloat16),
                w_ref[...].astype(jnp.bfloat16),
                preferred_element_type=jnp.float32,
            )

        def rs_rdma(r, s):
            return pltpu.make_async_remote_copy(
                src_ref=cb.at[r, s % 2],
                dst_ref=rb.at[r, s % 2],
                send_sem=send_sems.at[r, s],
                recv_sem=recv_sems.at[r, s],
                device_id=(downstream[r],),
                device_id_type=pl.DeviceIdType.MESH,
            )

        for s in range(N_DEV):
            cjs = [(me - s) % N_DEV, (me + s) % N_DEV]
            if s == 0:
                for r in (0, 1):
                    load_start(r, cjs[r])
            for r in (0, 1):
                load_wait(r, cjs[r])
            vals = [[conv_silu_dot(r, u) for u in range(CHUNK // SUB)]
                    for r in (0, 1)]
            if s < N_DEV - 1:
                nxt = [(me - s - 1) % N_DEV, (me + s + 1) % N_DEV]
                for r in (0, 1):
                    load_start(r, nxt[r])
            for r in (0, 1):
                if s >= 2:
                    rs_rdma(r, s - 2).wait_send()
                if s >= 1:
                    rs_rdma(r, s - 1).wait_recv()
                for u in range(CHUNK // SUB):
                    us = u * SUB
                    if s == 0:
                        cb[r, 0, us:us + SUB, :] = vals[r][u].astype(
                            jnp.bfloat16)
                    else:
                        cb[r, s % 2, us:us + SUB, :] = (
                            vals[r][u]
                            + rb[r, (s - 1) % 2, us:us + SUB, :].astype(
                                jnp.float32)
                        ).astype(jnp.bfloat16)
                if s == 1:
                    pl.semaphore_signal(
                        credit_sems.at[r], inc=1,
                        device_id=(upstream[r],),
                        device_id_type=pl.DeviceIdType.MESH,
                    )
                if s < N_DEV - 1:
                    if s == 2:
                        pl.semaphore_wait(credit_sems.at[r], 1)
                    rs_rdma(r, s).start()
        for r in (0, 1):
            rs_rdma(r, 2).wait_send()

        def ag_rdma(r, h):
            return pltpu.make_async_remote_copy(
                src_ref=cb.at[r, 1] if h == 0 else ab.at[r, h - 1],
                dst_ref=ab.at[r, h],
                send_sem=send_sems.at[r, 3 + h],
                recv_sem=recv_sems.at[r, 3 + h],
                device_id=(downstream[r],),
                device_id_type=pl.DeviceIdType.MESH,
            )

        stage_busy = [False, False]

        def stage_out(r, src_bf16, gid):
            if stage_busy[r]:
                pltpu.make_async_copy(
                    stage.at[r], out_ref.at[r, pl.ds(0, CHUNK), :],
                    stage_sems.at[r],
                ).wait()
            stage[r] = src_bf16[...].astype(jnp.float32)
            pltpu.make_async_copy(
                stage.at[r], out_ref.at[r, pl.ds(gid * CHUNK, CHUNK), :],
                stage_sems.at[r],
            ).start()
            stage_busy[r] = True

        fins = [(me + 1) % N_DEV, (me - 1) % N_DEV]
        for r in (0, 1):
            ag_rdma(r, 0).start()
        for r in (0, 1):
            stage_out(r, cb.at[r, 1], fins[r])
        for h in range(N_DEV - 1):
            gids = [(me - h) % N_DEV, (me + h) % N_DEV]
            for r in (0, 1):
                ag_rdma(r, h).wait_recv()
            if h < N_DEV - 2:
                for r in (0, 1):
                    ag_rdma(r, h + 1).start()
            for r in (0, 1):
                stage_out(r, ab.at[r, h], gids[r])

        for r in (0, 1):
            for h in range(N_DEV - 1):
                ag_rdma(r, h).wait_send()
            pltpu.make_async_copy(
                stage.at[r], out_ref.at[r, pl.ds(0, CHUNK), :],
                stage_sems.at[r],
            ).wait()

        @functools.partial(
            pl.run_scoped, second_barrier=pltpu.SemaphoreType.REGULAR
        )
        def _(second_barrier):
            for nbr in [left, right]:
                pl.semaphore_signal(
                    second_barrier, inc=1,
                    device_id=(nbr,), device_id_type=pl.DeviceIdType.MESH,
                )
            pl.semaphore_wait(second_barrier, 2)

    return pl.pallas_call(
        body,
        out_shape=jax.ShapeDtypeStruct((B, S, P), jnp.float32),
        in_specs=[
            pl.BlockSpec(memory_space=pl.ANY),
            pl.BlockSpec(memory_space=pltpu.VMEM),
            pl.BlockSpec(memory_space=pltpu.VMEM),
        ],
        out_specs=pl.BlockSpec(memory_space=pl.ANY),
        scratch_shapes=[
            pltpu.VMEM((2, CHUNK + HALO, C), jnp.float32),
            pltpu.VMEM((2, 2, CHUNK, P), jnp.bfloat16),
            pltpu.VMEM((2, 2, CHUNK, P), jnp.bfloat16),
            pltpu.VMEM((2, N_DEV - 1, CHUNK, P), jnp.bfloat16),
            pltpu.VMEM((2, CHUNK, P), jnp.float32),
            pltpu.SemaphoreType.DMA((2, 6)),
            pltpu.SemaphoreType.DMA((2, 6)),
            pltpu.SemaphoreType.DMA((4,)),
            pltpu.SemaphoreType.DMA((2,)),
            pltpu.SemaphoreType.REGULAR((2,)),
        ],
        compiler_params=pltpu.CompilerParams(
            collective_id=0,
            vmem_limit_bytes=60 * 1024 * 1024,
        ),
    )(x, k, Wp)


# device time: 181432 ns/iter; 1.1056x vs baseline; 1.1056x over previous
import functools
import os

import jax
import jax.numpy as jnp
from jax import lax
from jax.experimental import pallas as pl
from jax.experimental.pallas import tpu as pltpu

N_DEV = 4
_KMODE = os.environ.get("KMODE", "full")


def kernel(x, k, Wp):
    B, S, C = x.shape
    KT = k.shape[0]
    P = Wp.shape[1]
    CHUNK = S // N_DEV
    SUB = 512
    NSUB = CHUNK // SUB
    HALO = 8

    def body(x_ref, k_ref, w_ref, out_ref,
             xw, cb, rb, ab, stage,
             send_sems, recv_sems, copy_sems, stage_sems, credit_sems):
        me = lax.axis_index("i")
        left = (me - 1) % N_DEV
        right = (me + 1) % N_DEV
        downstream = [right, left]
        upstream = [left, right]

        barrier_sem = pltpu.get_barrier_semaphore()
        for nbr in [left, right]:
            pl.semaphore_signal(
                barrier_sem, inc=1,
                device_id=(nbr,), device_id_type=pl.DeviceIdType.MESH,
            )
        pl.semaphore_wait(barrier_sem, 2)

        def load_start(r, cj):
            lo = cj * CHUNK

            @pl.when(cj > 0)
            def _():
                pltpu.make_async_copy(
                    x_ref.at[r, pl.ds(lo - HALO, CHUNK + HALO), :], xw.at[r],
                    copy_sems.at[r],
                ).start()

            @pl.when(cj == 0)
            def _():
                pltpu.make_async_copy(
                    x_ref.at[r, pl.ds(0, CHUNK), :],
                    xw.at[r, pl.ds(HALO, CHUNK), :],
                    copy_sems.at[r],
                ).start()
                xw[r, 0:HALO, :] = jnp.zeros((HALO, C), jnp.float32)

        def load_wait(r, cj):
            lo = cj * CHUNK

            @pl.when(cj > 0)
            def _():
                pltpu.make_async_copy(
                    x_ref.at[r, pl.ds(lo - HALO, CHUNK + HALO), :], xw.at[r],
                    copy_sems.at[r],
                ).wait()

            @pl.when(cj == 0)
            def _():
                pltpu.make_async_copy(
                    x_ref.at[r, pl.ds(0, CHUNK), :],
                    xw.at[r, pl.ds(HALO, CHUNK), :],
                    copy_sems.at[r],
                ).wait()

        def conv_silu_dot(r, u):
            us = u * SUB + HALO
            acc = xw[r, us:us + SUB, :] * k_ref[KT - 1, :][None, :]
            for t in range(KT - 1):
                sh = KT - 1 - t
                acc += xw[r, us - sh:us + SUB - sh, :] * k_ref[t, :][None, :]
            a = acc * (1.0 / (1.0 + jnp.exp(-acc)))
            if _KMODE == "comm":
                return acc[:, :P]
            return jnp.dot(
                a.astype(jnp.bfloat16),
                w_ref[...].astype(jnp.bfloat16),
                preferred_element_type=jnp.float32,
            )

        sub = lambda u: pl.ds(u * SUB, SUB)

        def rs_sub(r, s, u):
            return pltpu.make_async_remote_copy(
                src_ref=cb.at[r, s % 2, sub(u), :],
                dst_ref=rb.at[r, s % 2, sub(u), :],
                send_sem=send_sems.at[r, NSUB * s + u],
                recv_sem=recv_sems.at[r, NSUB * s + u],
                device_id=(downstream[r],),
                device_id_type=pl.DeviceIdType.MESH,
            )

        stage_busy = [False, False]

        def stage_out(r, src_bf16, gid):
            if stage_busy[r]:
                pltpu.make_async_copy(
                    stage.at[r], out_ref.at[r, pl.ds(0, CHUNK), :],
                    stage_sems.at[r],
                ).wait()
            stage[r] = src_bf16[...].astype(jnp.float32)
            pltpu.make_async_copy(
                stage.at[r], out_ref.at[r, pl.ds(gid * CHUNK, CHUNK), :],
                stage_sems.at[r],
            ).start()
            stage_busy[r] = True

        for s in range(N_DEV):
            cjs = [(me - s) % N_DEV, (me + s) % N_DEV]
            if s == 0:
                for r in (0, 1):
                    load_start(r, cjs[r])
            for r in (0, 1):
                load_wait(r, cjs[r])
            for u in range(NSUB):
                vals = [conv_silu_dot(r, u) for r in (0, 1)]
                for r in (0, 1):
                    if _KMODE != "compute" and s >= 2:
                        rs_sub(r, s - 2, u).wait_send()
                    if _KMODE != "compute" and s >= 1:
                        rs_sub(r, s - 1, u).wait_recv()
                    if s == 0 or _KMODE == "compute":
                        cb[r, s % 2, sub(u), :] = vals[r].astype(jnp.bfloat16)
                    else:
                        cb[r, s % 2, sub(u), :] = (
                            vals[r]
                            + rb[r, (s - 1) % 2, sub(u), :].astype(
                                jnp.float32)
                        ).astype(jnp.bfloat16)
                    if _KMODE == "compute":
                        continue
                    if s == 1 and u == NSUB - 1:
                        pl.semaphore_signal(
                            credit_sems.at[r], inc=1,
                            device_id=(upstream[r],),
                            device_id_type=pl.DeviceIdType.MESH,
                        )
                    if s < N_DEV - 1:
                        if s == 2 and u == 0:
                            pl.semaphore_wait(credit_sems.at[r], 1)
                        rs_sub(r, s, u).start()
            if _KMODE == "compute":
                for r in (0, 1):
                    stage_out(r, cb.at[r, s % 2], cjs[r])
            if s < N_DEV - 1:
                nxt = [(me - s - 1) % N_DEV, (me + s + 1) % N_DEV]
                for r in (0, 1):
                    load_start(r, nxt[r])

        def ag_sub(r, h, u):
            return pltpu.make_async_remote_copy(
                src_ref=(cb.at[r, 1, sub(u), :] if h == 0
                         else ab.at[r, h - 1, sub(u), :]),
                dst_ref=ab.at[r, h, sub(u), :],
                send_sem=send_sems.at[r, NSUB * (N_DEV - 1 + h) + u],
                recv_sem=recv_sems.at[r, NSUB * (N_DEV - 1 + h) + u],
                device_id=(downstream[r],),
                device_id_type=pl.DeviceIdType.MESH,
            )

        if _KMODE != "compute":
            for r in (0, 1):
                for u in range(NSUB):
                    rs_sub(r, 2, u).wait_send()

            fins = [(me + 1) % N_DEV, (me - 1) % N_DEV]
            for u in range(NSUB):
                for r in (0, 1):
                    ag_sub(r, 0, u).start()
            for r in (0, 1):
                stage_out(r, cb.at[r, 1], fins[r])
            for h in range(N_DEV - 1):
                gids = [(me - h) % N_DEV, (me + h) % N_DEV]
                for u in range(NSUB):
                    for r in (0, 1):
                        ag_sub(r, h, u).wait_recv()
                    if h < N_DEV - 2:
                        for r in (0, 1):
                            ag_sub(r, h + 1, u).start()
                for r in (0, 1):
                    stage_out(r, ab.at[r, h], gids[r])

        for r in (0, 1):
            if _KMODE != "compute":
                for h in range(N_DEV - 1):
                    for u in range(NSUB):
                        ag_sub(r, h, u).wait_send()
            pltpu.make_async_copy(
                stage.at[r], out_ref.at[r, pl.ds(0, CHUNK), :],
                stage_sems.at[r],
            ).wait()

        @functools.partial(
            pl.run_scoped, second_barrier=pltpu.SemaphoreType.REGULAR
        )
        def _(second_barrier):
            for nbr in [left, right]:
                pl.semaphore_signal(
                    second_barrier, inc=1,
                    device_id=(nbr,), device_id_type=pl.DeviceIdType.MESH,
                )
            pl.semaphore_wait(second_barrier, 2)

    n_sems = 2 * (N_DEV - 1) * NSUB
    return pl.pallas_call(
        body,
        out_shape=jax.ShapeDtypeStruct((B, S, P), jnp.float32),
        in_specs=[
            pl.BlockSpec(memory_space=pl.ANY),
            pl.BlockSpec(memory_space=pltpu.VMEM),
            pl.BlockSpec(memory_space=pltpu.VMEM),
        ],
        out_specs=pl.BlockSpec(memory_space=pl.ANY),
        scratch_shapes=[
            pltpu.VMEM((2, CHUNK + HALO, C), jnp.float32),
            pltpu.VMEM((2, 2, CHUNK, P), jnp.bfloat16),
            pltpu.VMEM((2, 2, CHUNK, P), jnp.bfloat16),
            pltpu.VMEM((2, N_DEV - 1, CHUNK, P), jnp.bfloat16),
            pltpu.VMEM((2, CHUNK, P), jnp.float32),
            pltpu.SemaphoreType.DMA((2, n_sems)),
            pltpu.SemaphoreType.DMA((2, n_sems)),
            pltpu.SemaphoreType.DMA((2,)),
            pltpu.SemaphoreType.DMA((2,)),
            pltpu.SemaphoreType.REGULAR((2,)),
        ],
        compiler_params=pltpu.CompilerParams(
            collective_id=0,
            vmem_limit_bytes=60 * 1024 * 1024,
        ),
    )(x, k, Wp)
